# Initial kernel scaffold; baseline (speedup 1.0000x reference)
import jax
import jax.numpy as jnp
from jax.experimental import pallas as pl


def kernel(x, edge_index, edge_weight):
    raise NotImplementedError("write your pallas kernel here")


# speedup vs baseline: 4.5385x; 4.5385x over previous
import functools

import jax
import jax.numpy as jnp
from jax import lax
from jax.experimental import pallas as pl
from jax.experimental.pallas import tpu as pltpu
from jax.experimental.pallas import tpu_sc as plsc

N_NODES = 10000
N_EDGES = 320000
D_FEAT = 128
LANES = 16
DV = D_FEAT // LANES

NUM_CORES = 2
NUM_SUBCORES = 16
NUM_WORKERS = NUM_CORES * NUM_SUBCORES
E_PER_TILE = N_EDGES // NUM_WORKERS
CHUNK = 80
N_CHUNKS = E_PER_TILE // CHUNK
SLAB = 632
LAST_SLAB = N_NODES - SLAB * (NUM_SUBCORES - 1)
ZB = 40
NZ_FULL = SLAB // ZB
NZ_LAST = LAST_SLAB // ZB
ZREM = SLAB - NZ_FULL * ZB


def _bcast_lane(vec, lane):
    idx = jnp.full((LANES, 1), lane, dtype=jnp.int32)
    dnums = lax.GatherDimensionNumbers(
        offset_dims=(), collapsed_slice_dims=(0,), start_index_map=(0,))
    return lax.gather(vec, idx, dnums, (1,),
                      mode=lax.GatherScatterMode.PROMISE_IN_BOUNDS)


def _make_sc_kernel():
    mesh = plsc.VectorSubcoreMesh(core_axis_name="c", subcore_axis_name="s")

    @functools.partial(
        pl.kernel,
        mesh=mesh,
        out_type=[
            jax.ShapeDtypeStruct((N_NODES, D_FEAT), jnp.float32),
            jax.ShapeDtypeStruct((N_NODES, D_FEAT), jnp.float32),
        ],
        scratch_types=[
            pltpu.VMEM((CHUNK,), jnp.int32),
            pltpu.VMEM((CHUNK,), jnp.int32),
            pltpu.VMEM((CHUNK,), jnp.float32),
            pltpu.VMEM((CHUNK, D_FEAT), jnp.float32),
            pltpu.VMEM((ZB, D_FEAT), jnp.float32),
            pltpu.VMEM_SHARED((N_NODES, D_FEAT), jnp.float32),
            pltpu.SemaphoreType.DMA,
        ],
    )
    def spmm_kernel(x_hbm, col_hbm, row_hbm, w_hbm, out0, out1,
                    col_v, rowi_v, w_v, rows_v, zero_v, acc, sem):
        c = lax.axis_index("c")
        s = lax.axis_index("s")
        w_id = c * NUM_SUBCORES + s

        def zrow_body(r, _):
            for d in range(DV):
                zero_v[r, pl.ds(d * LANES, LANES)] = jnp.zeros(
                    (LANES,), jnp.float32)
            return _

        lax.fori_loop(0, ZB, zrow_body, None)
        slab_base = pl.multiple_of(s * SLAB, 8)
        last = NUM_SUBCORES - 1

        def zcopy_body(z, _):
            base = pl.multiple_of(slab_base + z * ZB, 8)
            pltpu.sync_copy(zero_v, acc.at[pl.ds(base, ZB)])
            return _

        lax.fori_loop(0, jnp.where(s == last, NZ_LAST, NZ_FULL),
                      zcopy_body, None)

        @pl.when(s < last)
        def _():
            base = pl.multiple_of(slab_base + NZ_FULL * ZB, 8)
            pltpu.sync_copy(zero_v.at[pl.ds(0, ZREM)],
                            acc.at[pl.ds(base, ZREM)])

        plsc.subcore_barrier()

        def chunk_body(k, _):
            e0 = w_id * E_PER_TILE + k * CHUNK
            pltpu.sync_copy(col_hbm.at[pl.ds(e0, CHUNK)], col_v)
            pltpu.sync_copy(row_hbm.at[pl.ds(e0, CHUNK)], rowi_v)
            pltpu.sync_copy(w_hbm.at[pl.ds(e0, CHUNK)], w_v)
            pltpu.async_copy(x_hbm.at[col_v], rows_v, sem).wait()

            def grp_body(g, _):
                wgrp = w_v[pl.ds(pl.multiple_of(g * LANES, LANES), LANES)]
                for ee in range(LANES):
                    wb = _bcast_lane(wgrp, ee)
                    e = g * LANES + ee
                    for d in range(DV):
                        sl = pl.ds(d * LANES, LANES)
                        rows_v[e, sl] = rows_v[e, sl] * wb
                return _

            lax.fori_loop(0, CHUNK // LANES, grp_body, None)
            pltpu.sync_copy(rows_v, acc.at[rowi_v], add=True)
            return _

        lax.fori_loop(0, N_CHUNKS, chunk_body, None)
        plsc.subcore_barrier()

        for core, out_ref in ((0, out0), (1, out1)):
            @pl.when(jnp.logical_and(c == core, s < NUM_SUBCORES - 1))
            def _(out_ref=out_ref):
                sl = pl.ds(slab_base, SLAB)
                pltpu.sync_copy(acc.at[sl], out_ref.at[sl])

            @pl.when(jnp.logical_and(c == core, s == NUM_SUBCORES - 1))
            def _(out_ref=out_ref):
                sl = pl.ds(slab_base, LAST_SLAB)
                pltpu.sync_copy(acc.at[sl], out_ref.at[sl])

    return spmm_kernel


_spmm = _make_sc_kernel()

_ADD_BLOCK = 1000


def _add_body(a_ref, b_ref, o_ref):
    o_ref[...] = a_ref[...] + b_ref[...]


def _combine(a, b):
    spec = pl.BlockSpec((_ADD_BLOCK, D_FEAT), lambda i: (i, 0))
    return pl.pallas_call(
        _add_body,
        grid=(N_NODES // _ADD_BLOCK,),
        in_specs=[spec, spec],
        out_specs=spec,
        out_shape=jax.ShapeDtypeStruct((N_NODES, D_FEAT), jnp.float32),
    )(a, b)


@jax.jit
def kernel(x, edge_index, edge_weight):
    col = edge_index[1].astype(jnp.int32)
    row = edge_index[0].astype(jnp.int32)
    w = edge_weight.astype(jnp.float32)
    out0, out1 = _spmm(x.astype(jnp.float32), col, row, w)
    return _combine(out0, out1)


# speedup vs baseline: 10.3485x; 2.2801x over previous
import functools

import jax
import jax.numpy as jnp
from jax import lax
from jax.experimental import pallas as pl
from jax.experimental.pallas import tpu as pltpu
from jax.experimental.pallas import tpu_sc as plsc

N_NODES = 10000
N_EDGES = 320000
D_FEAT = 128
LANES = 16
DV = D_FEAT // LANES

NUM_CORES = 2
NUM_SUBCORES = 16
NUM_WORKERS = NUM_CORES * NUM_SUBCORES
E_PER_TILE = N_EDGES // NUM_WORKERS
CHUNK = 80
N_CHUNKS = E_PER_TILE // CHUNK
NBUF = 3

SLAB = 632
LAST_SLAB = N_NODES - SLAB * (NUM_SUBCORES - 1)
ZB = 40
NZ_FULL = SLAB // ZB
NZ_LAST = LAST_SLAB // ZB
ZREM = SLAB - NZ_FULL * ZB


def _bcast_lane(vec, lane):
    idx = jnp.full((LANES, 1), lane, dtype=jnp.int32)
    dnums = lax.GatherDimensionNumbers(
        offset_dims=(), collapsed_slice_dims=(0,), start_index_map=(0,))
    return lax.gather(vec, idx, dnums, (1,),
                      mode=lax.GatherScatterMode.PROMISE_IN_BOUNDS)


def _make_sc_kernel():
    mesh = plsc.VectorSubcoreMesh(core_axis_name="c", subcore_axis_name="s")

    vmem_scratch = []
    for _ in range(NBUF):
        vmem_scratch += [
            pltpu.VMEM((CHUNK,), jnp.int32),
            pltpu.VMEM((CHUNK,), jnp.int32),
            pltpu.VMEM((CHUNK,), jnp.float32),
            pltpu.VMEM((CHUNK, D_FEAT), jnp.float32),
            pltpu.SemaphoreType.DMA,
            pltpu.SemaphoreType.DMA,
            pltpu.SemaphoreType.DMA,
        ]

    @functools.partial(
        pl.kernel,
        mesh=mesh,
        out_type=[
            jax.ShapeDtypeStruct((N_NODES, D_FEAT), jnp.float32),
            jax.ShapeDtypeStruct((N_NODES, D_FEAT), jnp.float32),
        ],
        scratch_types=vmem_scratch + [
            pltpu.VMEM((ZB, D_FEAT), jnp.float32),
            pltpu.VMEM_SHARED((N_NODES, D_FEAT), jnp.float32),
        ],
    )
    def spmm_kernel(x_hbm, col_hbm, row_hbm, w_hbm, out0, out1, *scratch):
        bufs = [scratch[7 * i:7 * (i + 1)] for i in range(NBUF)]
        zero_v, acc = scratch[7 * NBUF], scratch[7 * NBUF + 1]

        c = lax.axis_index("c")
        s = lax.axis_index("s")
        w_id = c * NUM_SUBCORES + s
        e_base = w_id * E_PER_TILE

        def fire_idx(k, bi):
            col_v, rowi_v, w_v, _, sem_i, _, _ = bufs[bi]
            e0 = pl.multiple_of(e_base + k * CHUNK, 8)
            pltpu.async_copy(col_hbm.at[pl.ds(e0, CHUNK)], col_v, sem_i)
            pltpu.async_copy(row_hbm.at[pl.ds(e0, CHUNK)], rowi_v, sem_i)
            pltpu.async_copy(w_hbm.at[pl.ds(e0, CHUNK)], w_v, sem_i)

        def wait_idx(bi):
            col_v, rowi_v, w_v, _, sem_i, _, _ = bufs[bi]
            pltpu.make_async_copy(col_hbm.at[pl.ds(0, CHUNK)], col_v,
                                  sem_i).wait()
            pltpu.make_async_copy(row_hbm.at[pl.ds(0, CHUNK)], rowi_v,
                                  sem_i).wait()
            pltpu.make_async_copy(w_hbm.at[pl.ds(0, CHUNK)], w_v,
                                  sem_i).wait()

        def fire_gather(bi):
            col_v, _, _, rows_v, _, sem_g, _ = bufs[bi]
            pltpu.async_copy(x_hbm.at[col_v], rows_v, sem_g)

        def wait_gather(bi):
            col_v, _, _, rows_v, _, sem_g, _ = bufs[bi]
            pltpu.make_async_copy(x_hbm.at[col_v], rows_v, sem_g).wait()

        def fire_scatter(bi):
            _, rowi_v, _, rows_v, _, _, sem_s = bufs[bi]
            pltpu.async_copy(rows_v, acc.at[rowi_v], sem_s, add=True)

        def wait_scatter(bi):
            _, rowi_v, _, rows_v, _, _, sem_s = bufs[bi]
            pltpu.make_async_copy(rows_v, acc.at[rowi_v], sem_s).wait()

        def compute(bi):
            _, _, w_v, rows_v, _, _, _ = bufs[bi]

            def grp_body(g, _):
                wgrp = w_v[pl.ds(pl.multiple_of(g * LANES, LANES), LANES)]
                for ee in range(LANES):
                    wb = _bcast_lane(wgrp, ee)
                    e = g * LANES + ee
                    for d in range(DV):
                        sl = pl.ds(d * LANES, LANES)
                        rows_v[e, sl] = rows_v[e, sl] * wb
                return _

            lax.fori_loop(0, CHUNK // LANES, grp_body, None)

        def zrow_body(r, _):
            for d in range(DV):
                zero_v[r, pl.ds(d * LANES, LANES)] = jnp.zeros(
                    (LANES,), jnp.float32)
            return _

        lax.fori_loop(0, ZB, zrow_body, None)
        slab_base = pl.multiple_of(s * SLAB, 8)
        last = NUM_SUBCORES - 1

        def zcopy_body(z, _):
            base = pl.multiple_of(slab_base + z * ZB, 8)
            pltpu.sync_copy(zero_v, acc.at[pl.ds(base, ZB)])
            return _

        lax.fori_loop(0, jnp.where(s == last, NZ_LAST, NZ_FULL),
                      zcopy_body, None)

        @pl.when(s < last)
        def _():
            base = pl.multiple_of(slab_base + NZ_FULL * ZB, 8)
            pltpu.sync_copy(zero_v.at[pl.ds(0, ZREM)],
                            acc.at[pl.ds(base, ZREM)])

        plsc.subcore_barrier()

        def body(k, bi, first=False, n_left=3):
            nbi = (bi + 1) % NBUF
            pbi = (bi + 2) % NBUF
            wait_gather(bi)
            if not first:
                wait_scatter(pbi)
            if n_left >= 2:
                wait_idx(nbi)
                fire_gather(nbi)
            if n_left >= 3:
                fire_idx(k + 2, pbi)
            compute(bi)
            fire_scatter(bi)

        fire_idx(0, 0)
        wait_idx(0)
        fire_gather(0)
        fire_idx(1, 1)

        body(0, 0, first=True)
        body(1, 1)
        body(2, 2)

        def main_body(j, _):
            for t in range(NBUF):
                body(3 * j + t, t)
            return _

        lax.fori_loop(1, 41, main_body, None)

        body(123, 0, n_left=2)
        body(124, 1, n_left=1)
        wait_scatter(1)

        plsc.subcore_barrier()

        for core, out_ref in ((0, out0), (1, out1)):
            @pl.when(jnp.logical_and(c == core, s < last))
            def _(out_ref=out_ref):
                sl = pl.ds(slab_base, SLAB)
                pltpu.sync_copy(acc.at[sl], out_ref.at[sl])

            @pl.when(jnp.logical_and(c == core, s == last))
            def _(out_ref=out_ref):
                sl = pl.ds(slab_base, LAST_SLAB)
                pltpu.sync_copy(acc.at[sl], out_ref.at[sl])

    return spmm_kernel


_spmm = _make_sc_kernel()

_ADD_BLOCK = 1000


def _add_body(a_ref, b_ref, o_ref):
    o_ref[...] = a_ref[...] + b_ref[...]


def _combine(a, b):
    spec = pl.BlockSpec((_ADD_BLOCK, D_FEAT), lambda i: (i, 0))
    return pl.pallas_call(
        _add_body,
        grid=(N_NODES // _ADD_BLOCK,),
        in_specs=[spec, spec],
        out_specs=spec,
        out_shape=jax.ShapeDtypeStruct((N_NODES, D_FEAT), jnp.float32),
    )(a, b)


@jax.jit
def kernel(x, edge_index, edge_weight):
    col = edge_index[1].astype(jnp.int32)
    row = edge_index[0].astype(jnp.int32)
    w = edge_weight.astype(jnp.float32)
    out0, out1 = _spmm(x.astype(jnp.float32), col, row, w)
    return _combine(out0, out1)


# speedup vs baseline: 12.2258x; 1.1814x over previous
import functools

import jax
import jax.numpy as jnp
from jax import lax
from jax.experimental import pallas as pl
from jax.experimental.pallas import tpu as pltpu
from jax.experimental.pallas import tpu_sc as plsc

N_NODES = 10000
N_EDGES = 320000
D_FEAT = 128
LANES = 16
DV = D_FEAT // LANES

NUM_CORES = 2
NUM_SUBCORES = 16
NUM_WORKERS = NUM_CORES * NUM_SUBCORES
E_PER_TILE = N_EDGES // NUM_WORKERS
CHUNK = 80
N_CHUNKS = E_PER_TILE // CHUNK
NBUF = 4

SLAB = 632
LAST_SLAB = N_NODES - SLAB * (NUM_SUBCORES - 1)
ZB = 40
NZ_FULL = SLAB // ZB
NZ_LAST = LAST_SLAB // ZB
ZREM = SLAB - NZ_FULL * ZB


def _bcast_lane(vec, lane):
    idx = jnp.full((LANES, 1), lane, dtype=jnp.int32)
    dnums = lax.GatherDimensionNumbers(
        offset_dims=(), collapsed_slice_dims=(0,), start_index_map=(0,))
    return lax.gather(vec, idx, dnums, (1,),
                      mode=lax.GatherScatterMode.PROMISE_IN_BOUNDS)


def _make_sc_kernel():
    mesh = plsc.VectorSubcoreMesh(core_axis_name="c", subcore_axis_name="s")

    vmem_scratch = []
    for _ in range(NBUF):
        vmem_scratch += [
            pltpu.VMEM((CHUNK,), jnp.int32),
            pltpu.VMEM((CHUNK,), jnp.int32),
            pltpu.VMEM((CHUNK,), jnp.float32),
            pltpu.VMEM((CHUNK, D_FEAT), jnp.float32),
            pltpu.SemaphoreType.DMA,
            pltpu.SemaphoreType.DMA,
            pltpu.SemaphoreType.DMA,
        ]

    @functools.partial(
        pl.kernel,
        mesh=mesh,
        out_type=[
            jax.ShapeDtypeStruct((N_NODES, D_FEAT), jnp.float32),
            jax.ShapeDtypeStruct((N_NODES, D_FEAT), jnp.float32),
        ],
        scratch_types=vmem_scratch + [
            pltpu.VMEM((ZB, D_FEAT), jnp.float32),
            pltpu.VMEM_SHARED((N_NODES, D_FEAT), jnp.float32),
        ],
    )
    def spmm_kernel(x_hbm, ei_hbm, w_hbm, out0, out1, *scratch):
        bufs = [scratch[7 * i:7 * (i + 1)] for i in range(NBUF)]
        zero_v, acc = scratch[7 * NBUF], scratch[7 * NBUF + 1]

        c = lax.axis_index("c")
        s = lax.axis_index("s")
        w_id = c * NUM_SUBCORES + s
        e_base = w_id * E_PER_TILE

        def fire_idx(k, bi):
            col_v, rowi_v, w_v, _, sem_i, _, _ = bufs[bi]
            e0 = pl.multiple_of(e_base + k * CHUNK, 8)
            c0 = pl.multiple_of(N_EDGES + e_base + k * CHUNK, 8)
            pltpu.async_copy(ei_hbm.at[pl.ds(e0, CHUNK)], rowi_v, sem_i)
            pltpu.async_copy(ei_hbm.at[pl.ds(c0, CHUNK)], col_v, sem_i)
            pltpu.async_copy(w_hbm.at[pl.ds(e0, CHUNK)], w_v, sem_i)

        def wait_idx(bi):
            col_v, rowi_v, w_v, _, sem_i, _, _ = bufs[bi]
            pltpu.make_async_copy(ei_hbm.at[pl.ds(0, CHUNK)], rowi_v,
                                  sem_i).wait()
            pltpu.make_async_copy(ei_hbm.at[pl.ds(0, CHUNK)], col_v,
                                  sem_i).wait()
            pltpu.make_async_copy(w_hbm.at[pl.ds(0, CHUNK)], w_v,
                                  sem_i).wait()

        def fire_gather(bi):
            col_v, _, _, rows_v, _, sem_g, _ = bufs[bi]
            pltpu.async_copy(x_hbm.at[col_v], rows_v, sem_g)

        def wait_gather(bi):
            col_v, _, _, rows_v, _, sem_g, _ = bufs[bi]
            pltpu.make_async_copy(x_hbm.at[col_v], rows_v, sem_g).wait()

        def fire_scatter(bi):
            _, rowi_v, _, rows_v, _, _, sem_s = bufs[bi]
            pltpu.async_copy(rows_v, acc.at[rowi_v], sem_s, add=True)

        def wait_scatter(bi):
            _, rowi_v, _, rows_v, _, _, sem_s = bufs[bi]
            pltpu.make_async_copy(rows_v, acc.at[rowi_v], sem_s).wait()

        def compute(bi):
            _, _, w_v, rows_v, _, _, _ = bufs[bi]

            def grp_body(g, _):
                wgrp = w_v[pl.ds(pl.multiple_of(g * LANES, LANES), LANES)]
                for ee in range(LANES):
                    wb = _bcast_lane(wgrp, ee)
                    e = g * LANES + ee
                    for d in range(DV):
                        sl = pl.ds(d * LANES, LANES)
                        rows_v[e, sl] = rows_v[e, sl] * wb
                return _

            lax.fori_loop(0, CHUNK // LANES, grp_body, None)

        def zrow_body(r, _):
            for d in range(DV):
                zero_v[r, pl.ds(d * LANES, LANES)] = jnp.zeros(
                    (LANES,), jnp.float32)
            return _

        lax.fori_loop(0, ZB, zrow_body, None)
        slab_base = pl.multiple_of(s * SLAB, 8)
        last = NUM_SUBCORES - 1

        def zcopy_body(z, _):
            base = pl.multiple_of(slab_base + z * ZB, 8)
            pltpu.sync_copy(zero_v, acc.at[pl.ds(base, ZB)])
            return _

        lax.fori_loop(0, jnp.where(s == last, NZ_LAST, NZ_FULL),
                      zcopy_body, None)

        @pl.when(s < last)
        def _():
            base = pl.multiple_of(slab_base + NZ_FULL * ZB, 8)
            pltpu.sync_copy(zero_v.at[pl.ds(0, ZREM)],
                            acc.at[pl.ds(base, ZREM)])

        plsc.subcore_barrier()

        def body(k, bi, first=False, n_left=4):
            gnb = (bi + 2) % NBUF
            inb = (bi + 3) % NBUF
            if n_left >= 3:
                wait_idx(gnb)
                fire_gather(gnb)
            wait_gather(bi)
            compute(bi)
            fire_scatter(bi)
            if not first:
                wait_scatter(inb)
            if n_left >= 4:
                fire_idx(k + 3, inb)

        fire_idx(0, 0)
        fire_idx(1, 1)
        fire_idx(2, 2)
        wait_idx(0)
        fire_gather(0)
        wait_idx(1)
        fire_gather(1)

        body(0, 0, first=True)
        body(1, 1)

        def main_body(j, _):
            for t in range(NBUF):
                body(2 + 4 * j + t, (2 + t) % NBUF)
            return _

        lax.fori_loop(0, 30, main_body, None)

        body(122, 2, n_left=3)
        body(123, 3, n_left=2)
        body(124, 0, n_left=1)
        wait_scatter(0)

        plsc.subcore_barrier()

        for core, out_ref in ((0, out0), (1, out1)):
            @pl.when(jnp.logical_and(c == core, s < last))
            def _(out_ref=out_ref):
                sl = pl.ds(slab_base, SLAB)
                pltpu.sync_copy(acc.at[sl], out_ref.at[sl])

            @pl.when(jnp.logical_and(c == core, s == last))
            def _(out_ref=out_ref):
                sl = pl.ds(slab_base, LAST_SLAB)
                pltpu.sync_copy(acc.at[sl], out_ref.at[sl])

    return spmm_kernel


_spmm = _make_sc_kernel()

_ADD_BLOCK = 1000


def _add_body(a_ref, b_ref, o_ref):
    o_ref[...] = a_ref[...] + b_ref[...]


def _combine(a, b):
    spec = pl.BlockSpec((_ADD_BLOCK, D_FEAT), lambda i: (i, 0))
    return pl.pallas_call(
        _add_body,
        grid=(N_NODES // _ADD_BLOCK,),
        in_specs=[spec, spec],
        out_specs=spec,
        out_shape=jax.ShapeDtypeStruct((N_NODES, D_FEAT), jnp.float32),
    )(a, b)


@jax.jit
def kernel(x, edge_index, edge_weight):
    out0, out1 = _spmm(x.astype(jnp.float32),
                       edge_index.astype(jnp.int32).reshape(-1),
                       edge_weight.astype(jnp.float32))
    return _combine(out0, out1)


# speedup vs baseline: 12.8081x; 1.0476x over previous
import functools

import jax
import jax.numpy as jnp
from jax import lax
from jax.experimental import pallas as pl
from jax.experimental.pallas import tpu as pltpu
from jax.experimental.pallas import tpu_sc as plsc

N_NODES = 10000
N_EDGES = 320000
D_FEAT = 128
LANES = 16
DV = D_FEAT // LANES

NUM_CORES = 2
NUM_SUBCORES = 16
NUM_WORKERS = NUM_CORES * NUM_SUBCORES
E_PER_TILE = N_EDGES // NUM_WORKERS
CHUNK = 80
N_CHUNKS = E_PER_TILE // CHUNK
NRING = 3
BLK = 24
NBLK = N_CHUNKS // BLK
REM_CHUNKS = N_CHUNKS - NBLK * BLK
BLK_E = BLK * CHUNK
REM_E = REM_CHUNKS * CHUNK

SLAB = 632
LAST_SLAB = N_NODES - SLAB * (NUM_SUBCORES - 1)


def _bcast_lane(vec, lane):
    idx = jnp.full((LANES, 1), lane, dtype=jnp.int32)
    dnums = lax.GatherDimensionNumbers(
        offset_dims=(), collapsed_slice_dims=(0,), start_index_map=(0,))
    return lax.gather(vec, idx, dnums, (1,),
                      mode=lax.GatherScatterMode.PROMISE_IN_BOUNDS)


def _make_sc_kernel():
    mesh = plsc.VectorSubcoreMesh(core_axis_name="c", subcore_axis_name="s")

    scratch = (
        [pltpu.VMEM((BLK_E,), jnp.int32) for _ in range(2)]
        + [pltpu.VMEM((BLK_E,), jnp.int32) for _ in range(2)]
        + [pltpu.VMEM((BLK_E,), jnp.float32) for _ in range(2)]
        + [pltpu.SemaphoreType.DMA for _ in range(2)]
        + [pltpu.VMEM((CHUNK, D_FEAT), jnp.float32)
           for _ in range(NRING)]
        + [pltpu.VMEM((CHUNK,), jnp.int32) for _ in range(NRING)]
        + [pltpu.SemaphoreType.DMA for _ in range(NRING)]
        + [pltpu.SemaphoreType.DMA for _ in range(NRING)]
        + [pltpu.VMEM_SHARED((N_NODES, D_FEAT), jnp.float32)]
    )

    @functools.partial(
        pl.kernel,
        mesh=mesh,
        out_type=[
            jax.ShapeDtypeStruct((N_NODES, D_FEAT), jnp.float32),
            jax.ShapeDtypeStruct((N_NODES, D_FEAT), jnp.float32),
        ],
        scratch_types=scratch,
    )
    def spmm_kernel(x_hbm, ei_hbm, w_hbm, out0, out1,
                    cb0, cb1, rb0, rb1, wb0, wb1, sb0, sb1,
                    rv0, rv1, rv2, st0, st1, st2,
                    sg0, sg1, sg2, ss0, ss1, ss2, acc):
        colb, rowb, wblk, semb = (cb0, cb1), (rb0, rb1), (wb0, wb1), (sb0, sb1)
        rows = (rv0, rv1, rv2)
        stg = (st0, st1, st2)
        sem_g = (sg0, sg1, sg2)
        sem_s = (ss0, ss1, ss2)

        c = lax.axis_index("c")
        s = lax.axis_index("s")
        w_id = c * NUM_SUBCORES + s
        e_base = w_id * E_PER_TILE

        def fire_block(b, ne=BLK_E):
            p = b % 2
            e0 = pl.multiple_of(e_base + b * BLK_E, 8)
            c0 = pl.multiple_of(N_EDGES + e_base + b * BLK_E, 8)
            pltpu.async_copy(ei_hbm.at[pl.ds(e0, ne)],
                             rowb[p].at[pl.ds(0, ne)], semb[p])
            pltpu.async_copy(ei_hbm.at[pl.ds(c0, ne)],
                             colb[p].at[pl.ds(0, ne)], semb[p])
            pltpu.async_copy(w_hbm.at[pl.ds(e0, ne)],
                             wblk[p].at[pl.ds(0, ne)], semb[p])

        def wait_block(b, ne=BLK_E):
            p = b % 2
            pltpu.make_async_copy(ei_hbm.at[pl.ds(0, ne)],
                                  rowb[p].at[pl.ds(0, ne)], semb[p]).wait()
            pltpu.make_async_copy(ei_hbm.at[pl.ds(0, ne)],
                                  colb[p].at[pl.ds(0, ne)], semb[p]).wait()
            pltpu.make_async_copy(w_hbm.at[pl.ds(0, ne)],
                                  wblk[p].at[pl.ds(0, ne)], semb[p]).wait()

        def fire_gather(t, blk, o):
            col = colb[blk].at[pl.ds(pl.multiple_of(o * CHUNK, 16), CHUNK)]
            pltpu.async_copy(x_hbm.at[col], rows[t], sem_g[t])

        def wait_gather(t):
            col = colb[0].at[pl.ds(0, CHUNK)]
            pltpu.make_async_copy(x_hbm.at[col], rows[t], sem_g[t]).wait()

        def fire_scatter(t):
            pltpu.async_copy(rows[t], acc.at[stg[t]], sem_s[t], add=True)

        def wait_scatter(t):
            pltpu.make_async_copy(rows[t], acc.at[stg[t]], sem_s[t]).wait()

        def compute(t, blk, o):
            base = pl.multiple_of(o * CHUNK, 16)
            for g in range(CHUNK // LANES):
                stg[t][pl.ds(g * LANES, LANES)] = (
                    rowb[blk][pl.ds(base + g * LANES, LANES)])

            def grp_body(g, _):
                woff = pl.multiple_of(base + g * LANES, LANES)
                wgrp = wblk[blk][pl.ds(woff, LANES)]
                for ee in range(LANES):
                    wb = _bcast_lane(wgrp, ee)
                    e = g * LANES + ee
                    for d in range(DV):
                        sl = pl.ds(d * LANES, LANES)
                        rows[t][e, sl] = rows[t][e, sl] * wb
                return _

            lax.fori_loop(0, CHUNK // LANES, grp_body, None)

        def body(t, blk, o, nxt=None, first=False):
            pt = (t + 2) % NRING
            wait_gather(t)
            if not first:
                wait_scatter(pt)
            if nxt is not None:
                fire_gather(pt, nxt[0], nxt[1])
            compute(t, blk, o)
            fire_scatter(t)

        def zrow_body(r, _):
            for d in range(DV):
                rows[0][r, pl.ds(d * LANES, LANES)] = jnp.zeros(
                    (LANES,), jnp.float32)
            return _

        lax.fori_loop(0, CHUNK, zrow_body, None)
        slab_base = pl.multiple_of(s * SLAB, 8)
        last = NUM_SUBCORES - 1

        def zcopy_body(z, _):
            base = pl.multiple_of(slab_base + z * CHUNK, 8)
            pltpu.sync_copy(rows[0], acc.at[pl.ds(base, CHUNK)])
            return _

        lax.fori_loop(0, jnp.where(s == last, 6, 7), zcopy_body, None)

        @pl.when(s < last)
        def _():
            base = pl.multiple_of(slab_base + 7 * CHUNK, 8)
            pltpu.sync_copy(rows[0].at[pl.ds(0, 72)],
                            acc.at[pl.ds(base, 72)])

        @pl.when(s == last)
        def _():
            base = pl.multiple_of(slab_base + 6 * CHUNK, 8)
            pltpu.sync_copy(rows[0].at[pl.ds(0, 40)],
                            acc.at[pl.ds(base, 40)])

        plsc.subcore_barrier()

        fire_block(0)
        wait_block(0)
        fire_block(1)
        fire_gather(0, 0, 0)
        fire_gather(1, 0, 1)

        for b in range(NBLK):
            p = b % 2
            np_ = (b + 1) % 2
            if b >= 1:
                fire_block(b + 1, BLK_E if b + 1 < NBLK else REM_E)

            if b == 0:
                for t in range(NRING):
                    body(t, p, t, nxt=(p, t + 2), first=(t == 0))
                i_lo = 1
            else:
                i_lo = 0

            def tri_body(i, _, p=p):
                for t in range(NRING):
                    o = 3 * i + t
                    body(t, p, o, nxt=(p, o + 2))
                return _

            lax.fori_loop(i_lo, (BLK // 3) - 1, tri_body, None)

            wait_block(b + 1, BLK_E if b + 1 < NBLK else REM_E)

            for t in range(NRING):
                o = BLK - 3 + t
                nxt = (p, o + 2) if o + 2 < BLK else (np_, o + 2 - BLK)
                body(t, p, o, nxt=nxt)

        rp = NBLK % 2
        for j in range(REM_CHUNKS):
            t = j % NRING
            nxt = (rp, j + 2) if j + 2 < REM_CHUNKS else None
            body(t, rp, j, nxt=nxt)
        wait_scatter((REM_CHUNKS - 1) % NRING)

        plsc.subcore_barrier()

        for core, out_ref in ((0, out0), (1, out1)):
            @pl.when(jnp.logical_and(c == core, s < last))
            def _(out_ref=out_ref):
                sl = pl.ds(slab_base, SLAB)
                pltpu.sync_copy(acc.at[sl], out_ref.at[sl])

            @pl.when(jnp.logical_and(c == core, s == last))
            def _(out_ref=out_ref):
                sl = pl.ds(slab_base, LAST_SLAB)
                pltpu.sync_copy(acc.at[sl], out_ref.at[sl])

    return spmm_kernel


_spmm = _make_sc_kernel()

_ADD_BLOCK = 1000


def _add_body(a_ref, b_ref, o_ref):
    o_ref[...] = a_ref[...] + b_ref[...]


def _combine(a, b):
    spec = pl.BlockSpec((_ADD_BLOCK, D_FEAT), lambda i: (i, 0))
    return pl.pallas_call(
        _add_body,
        grid=(N_NODES // _ADD_BLOCK,),
        in_specs=[spec, spec],
        out_specs=spec,
        out_shape=jax.ShapeDtypeStruct((N_NODES, D_FEAT), jnp.float32),
    )(a, b)


@jax.jit
def kernel(x, edge_index, edge_weight):
    out0, out1 = _spmm(x.astype(jnp.float32),
                       edge_index.astype(jnp.int32).reshape(-1),
                       edge_weight.astype(jnp.float32))
    return _combine(out0, out1)
